# baseline (speedup 1.0000x reference)
import functools
import math

import jax
import jax.numpy as jnp
from jax import lax
from jax.experimental import pallas as pl
from jax.experimental.pallas import tpu as pltpu
from jax.experimental.pallas import tpu_sc as plsc

_K = 4
_SECOND_K = 4
_NIDX = _K + _SECOND_K
_BATCH = 16384
_LEVEL_SQ_SUM = math.exp(math.log(1e24) / _NIDX)
_SCALE_NUM = _LEVEL_SQ_SUM ** (_NIDX / 2.0)

_NC, _NS, _L = 2, 16, 16
_NW = _NC * _NS
_RPW = _BATCH // _NW


def _gather_prod_sc(idx_t, first_mat, second_mat):
    mesh = plsc.VectorSubcoreMesh(core_axis_name="c", subcore_axis_name="s")

    @functools.partial(
        pl.kernel,
        out_type=jax.ShapeDtypeStruct((_BATCH,), jnp.float32),
        mesh=mesh,
        scratch_types=[
            [pltpu.VMEM((_RPW,), jnp.int32) for _ in range(_NIDX)],
            [pltpu.VMEM((_RPW,), jnp.float32) for _ in range(_NIDX)],
            pltpu.VMEM((_RPW,), jnp.float32),
            pltpu.SemaphoreType.DMA,
            pltpu.SemaphoreType.DMA,
        ],
    )
    def k(idx_hbm, fm_hbm, sm_hbm, out_hbm, idxvs, valsvs, prodv, sem, isem):
        wid = lax.axis_index("s") * _NC + lax.axis_index("c")
        base = wid * _RPW
        icopies = [
            pltpu.async_copy(idx_hbm.at[j, pl.ds(base, _RPW)], idxvs[j], isem)
            for j in range(_NIDX)
        ]
        copies = []
        for j in range(_NIDX):
            icopies[j].wait()
            tbl = fm_hbm if j < _K else sm_hbm
            copies.append(pltpu.async_copy(tbl.at[idxvs[j]], valsvs[j], sem))
        for cp in copies:
            cp.wait()
        for i in range(_RPW // _L):
            sl = pl.ds(i * _L, _L)
            p = valsvs[0][sl]
            for j in range(1, _NIDX):
                p = p * valsvs[j][sl]
            prodv[sl] = p
        pltpu.sync_copy(prodv, out_hbm.at[pl.ds(base, _RPW)])

    return k(idx_t, first_mat, second_mat)


def _scale_tc(fm2d, sm2d):
    def body(fm_ref, sm_ref, out_ref):
        fm = fm_ref[...]
        sm = sm_ref[...]
        s1 = jnp.sum(fm * fm)
        s2 = jnp.sum(sm * sm)
        scale = _SCALE_NUM / ((s1 * s1) * (s2 * s2))
        out_ref[...] = jnp.broadcast_to(scale, (1, 1))

    return pl.pallas_call(
        body,
        out_shape=jax.ShapeDtypeStruct((1, 1), jnp.float32),
    )(fm2d, sm2d)


def _combine_tc(g2d, scale2d):
    def body(g_ref, s_ref, out_ref):
        out_ref[...] = g_ref[...] * s_ref[0, 0]

    return pl.pallas_call(
        body,
        out_shape=jax.ShapeDtypeStruct(g2d.shape, jnp.float32),
    )(g2d, scale2d)


def kernel(_input, first_mat, second_mat):
    idx_t = _input.astype(jnp.int32).T
    g = _gather_prod_sc(idx_t, first_mat, second_mat)
    scale = _scale_tc(first_mat.reshape(1000, 1000),
                      second_mat.reshape(1000, 1000))
    out2 = _combine_tc(g.reshape(128, 128), scale)
    return out2.reshape(_BATCH)


# speedup vs baseline: 1.2808x; 1.2808x over previous
import functools
import math

import jax
import jax.numpy as jnp
from jax import lax
from jax.experimental import pallas as pl
from jax.experimental.pallas import tpu as pltpu
from jax.experimental.pallas import tpu_sc as plsc

_K = 4
_SECOND_K = 4
_NIDX = _K + _SECOND_K
_BATCH = 16384
_TBL = 1000000
_LEVEL_SQ_SUM = math.exp(math.log(1e24) / _NIDX)
_SCALE_NUM = _LEVEL_SQ_SUM ** (_NIDX / 2.0)

_NC, _NS, _L = 2, 16, 16
_NW = _NC * _NS
_RPW = _BATCH // _NW

_CHUNK = 124928
_NCHUNK = 8
_MAIN = _CHUNK * _NCHUNK
_TAIL = _TBL - _MAIN


def _gather_prod_sc(idx, first_mat, second_mat):
    mesh = plsc.VectorSubcoreMesh(core_axis_name="c", subcore_axis_name="s")

    @functools.partial(
        pl.kernel,
        out_type=(jax.ShapeDtypeStruct((_BATCH,), jnp.float32),
                  jax.ShapeDtypeStruct((2 * _L,), jnp.float32)),
        mesh=mesh,
        scratch_types=[
            [pltpu.VMEM((_RPW,), jnp.int32) for _ in range(_NIDX)],
            [pltpu.VMEM((_RPW,), jnp.float32) for _ in range(_NIDX)],
            pltpu.VMEM((_RPW,), jnp.float32),
            pltpu.VMEM((_TAIL,), jnp.float32),
            pltpu.VMEM((_L,), jnp.float32),
            pltpu.SemaphoreType.DMA,
            pltpu.SemaphoreType.DMA,
        ],
    )
    def k(idx_hbm, fm_hbm, sm_hbm, out_hbm, tails_hbm,
          idxvs, valsvs, prodv, tailv, tpart, sem, isem):
        wid = lax.axis_index("s") * _NC + lax.axis_index("c")
        base = wid * _RPW
        icopies = [
            pltpu.async_copy(idx_hbm.at[j, pl.ds(base, _RPW)], idxvs[j], isem)
            for j in range(_NIDX)
        ]
        copies = []
        for j in range(_NIDX):
            icopies[j].wait()
            tbl = fm_hbm if j < _K else sm_hbm
            copies.append(pltpu.async_copy(tbl.at[idxvs[j]], valsvs[j], sem))

        for w, tbl in ((0, fm_hbm), (1, sm_hbm)):
            @pl.when(wid == w)
            def _():
                pltpu.sync_copy(tbl.at[pl.ds(_MAIN, _TAIL)], tailv)
                acc = tailv[pl.ds(0, _L)]
                acc = acc * acc
                for t in range(1, _TAIL // _L):
                    v = tailv[pl.ds(t * _L, _L)]
                    acc = acc + v * v
                tpart[...] = acc
                pltpu.sync_copy(tpart, tails_hbm.at[pl.ds(w * _L, _L)])

        for cp in copies:
            cp.wait()
        for i in range(_RPW // _L):
            sl = pl.ds(i * _L, _L)
            p = valsvs[0][sl]
            for j in range(1, _NIDX):
                p = p * valsvs[j][sl]
            prodv[sl] = p
        pltpu.sync_copy(prodv, out_hbm.at[pl.ds(base, _RPW)])

    return k(idx, first_mat, second_mat)


def _sumsq_tc(fm, sm):
    def body(fm_hbm, sm_hbm, out_ref, fmv, smv, sem1, sem2):
        c1 = [pltpu.make_async_copy(fm_hbm.at[pl.ds(r * _CHUNK, _CHUNK)],
                                    fmv.at[r], sem1) for r in range(_NCHUNK)]
        c2 = [pltpu.make_async_copy(sm_hbm.at[pl.ds(r * _CHUNK, _CHUNK)],
                                    smv.at[r], sem2) for r in range(_NCHUNK)]
        for c in c1 + c2:
            c.start()
        for c in c1:
            c.wait()
        x = fmv[...]
        s1 = jnp.sum(x * x)
        for c in c2:
            c.wait()
        y = smv[...]
        s2 = jnp.sum(y * y)
        out_ref[...] = jnp.concatenate(
            [jnp.broadcast_to(s1, (1, 1)), jnp.broadcast_to(s2, (1, 1))], axis=1)

    return pl.pallas_call(
        body,
        in_specs=[pl.BlockSpec(memory_space=pltpu.MemorySpace.HBM),
                  pl.BlockSpec(memory_space=pltpu.MemorySpace.HBM)],
        out_specs=pl.BlockSpec(memory_space=pltpu.VMEM),
        out_shape=jax.ShapeDtypeStruct((1, 2), jnp.float32),
        scratch_shapes=[pltpu.VMEM((_NCHUNK, _CHUNK), jnp.float32),
                        pltpu.VMEM((_NCHUNK, _CHUNK), jnp.float32),
                        pltpu.SemaphoreType.DMA,
                        pltpu.SemaphoreType.DMA],
    )(fm, sm)


def _combine_tc(g, smain, tails):
    def body(g_ref, s_ref, t_ref, out_ref):
        s1 = s_ref[0, 0] + jnp.sum(t_ref[pl.ds(0, _L)])
        s2 = s_ref[0, 1] + jnp.sum(t_ref[pl.ds(_L, _L)])
        scale = _SCALE_NUM / ((s1 * s1) * (s2 * s2))
        out_ref[...] = g_ref[...] * scale

    return pl.pallas_call(
        body,
        out_shape=jax.ShapeDtypeStruct(g.shape, jnp.float32),
    )(g, smain, tails)


def kernel(_input, first_mat, second_mat):
    idx = _input.astype(jnp.int32).T
    g, tails = _gather_prod_sc(idx, first_mat, second_mat)
    smain = _sumsq_tc(first_mat, second_mat)
    return _combine_tc(g, smain, tails)


# speedup vs baseline: 1.2864x; 1.0044x over previous
import functools
import math

import jax
import jax.numpy as jnp
from jax import lax
from jax.experimental import pallas as pl
from jax.experimental.pallas import tpu as pltpu
from jax.experimental.pallas import tpu_sc as plsc

_K = 4
_SECOND_K = 4
_NIDX = _K + _SECOND_K
_BATCH = 16384
_TBL = 1000000
_LEVEL_SQ_SUM = math.exp(math.log(1e24) / _NIDX)
_SCALE_NUM = _LEVEL_SQ_SUM ** (_NIDX / 2.0)

_NC, _NS, _L = 2, 16, 16
_NW = _NC * _NS
_RPW = _BATCH // _NW

_CHUNK = 124928
_NCHUNK = 8
_MAIN = _CHUNK * _NCHUNK
_TAIL = _TBL - _MAIN


def _gather_prod_sc(idx, first_mat, second_mat):
    mesh = plsc.VectorSubcoreMesh(core_axis_name="c", subcore_axis_name="s")

    @functools.partial(
        pl.kernel,
        out_type=(jax.ShapeDtypeStruct((_BATCH,), jnp.float32),
                  jax.ShapeDtypeStruct((2 * _L,), jnp.float32)),
        mesh=mesh,
        scratch_types=[
            [pltpu.VMEM((_RPW,), jnp.int32) for _ in range(_NIDX)],
            [pltpu.VMEM((_RPW,), jnp.float32) for _ in range(_NIDX)],
            pltpu.VMEM((_RPW,), jnp.float32),
            pltpu.VMEM((_TAIL,), jnp.float32),
            pltpu.VMEM((_L,), jnp.float32),
            pltpu.SemaphoreType.DMA,
            pltpu.SemaphoreType.DMA,
        ],
    )
    def k(idx_hbm, fm_hbm, sm_hbm, out_hbm, tails_hbm,
          idxvs, valsvs, prodv, tailv, tpart, sem, isem):
        wid = lax.axis_index("s") * _NC + lax.axis_index("c")
        base = wid * _RPW
        icopies = [
            pltpu.async_copy(idx_hbm.at[j, pl.ds(base, _RPW)], idxvs[j], isem)
            for j in range(_NIDX)
        ]
        copies = []
        for j in range(_NIDX):
            icopies[j].wait()
            tbl = fm_hbm if j < _K else sm_hbm
            copies.append(pltpu.async_copy(tbl.at[idxvs[j]], valsvs[j], sem))

        for w, tbl in ((0, fm_hbm), (1, sm_hbm)):
            @pl.when(wid == w)
            def _():
                pltpu.sync_copy(tbl.at[pl.ds(_MAIN, _TAIL)], tailv)
                acc = tailv[pl.ds(0, _L)]
                acc = acc * acc
                for t in range(1, _TAIL // _L):
                    v = tailv[pl.ds(t * _L, _L)]
                    acc = acc + v * v
                tpart[...] = acc
                pltpu.sync_copy(tpart, tails_hbm.at[pl.ds(w * _L, _L)])

        copies[0].wait()
        copies[1].wait()
        for i in range(_RPW // _L):
            sl = pl.ds(i * _L, _L)
            prodv[sl] = valsvs[0][sl] * valsvs[1][sl]
        for j in range(2, _NIDX, 2):
            copies[j].wait()
            copies[j + 1].wait()
            for i in range(_RPW // _L):
                sl = pl.ds(i * _L, _L)
                prodv[sl] = prodv[sl] * (valsvs[j][sl] * valsvs[j + 1][sl])
        pltpu.sync_copy(prodv, out_hbm.at[pl.ds(base, _RPW)])

    return k(idx, first_mat, second_mat)


def _sumsq_tc(fm, sm):
    def body(fm_hbm, sm_hbm, out_ref, fmv, smv, sem1, sem2):
        c1 = [pltpu.make_async_copy(fm_hbm.at[pl.ds(r * _CHUNK, _CHUNK)],
                                    fmv.at[r], sem1) for r in range(_NCHUNK)]
        c2 = [pltpu.make_async_copy(sm_hbm.at[pl.ds(r * _CHUNK, _CHUNK)],
                                    smv.at[r], sem2) for r in range(_NCHUNK)]
        for c in c1 + c2:
            c.start()
        for c in c1:
            c.wait()
        x = fmv[...]
        s1 = jnp.sum(x * x)
        for c in c2:
            c.wait()
        y = smv[...]
        s2 = jnp.sum(y * y)
        out_ref[...] = jnp.concatenate(
            [jnp.broadcast_to(s1, (1, 1)), jnp.broadcast_to(s2, (1, 1))], axis=1)

    return pl.pallas_call(
        body,
        in_specs=[pl.BlockSpec(memory_space=pltpu.MemorySpace.HBM),
                  pl.BlockSpec(memory_space=pltpu.MemorySpace.HBM)],
        out_specs=pl.BlockSpec(memory_space=pltpu.VMEM),
        out_shape=jax.ShapeDtypeStruct((1, 2), jnp.float32),
        scratch_shapes=[pltpu.VMEM((_NCHUNK, _CHUNK), jnp.float32),
                        pltpu.VMEM((_NCHUNK, _CHUNK), jnp.float32),
                        pltpu.SemaphoreType.DMA,
                        pltpu.SemaphoreType.DMA],
    )(fm, sm)


def _combine_tc(g, smain, tails):
    def body(g_ref, s_ref, t_ref, out_ref):
        s1 = s_ref[0, 0] + jnp.sum(t_ref[pl.ds(0, _L)])
        s2 = s_ref[0, 1] + jnp.sum(t_ref[pl.ds(_L, _L)])
        scale = _SCALE_NUM / ((s1 * s1) * (s2 * s2))
        out_ref[...] = g_ref[...] * scale

    return pl.pallas_call(
        body,
        out_shape=jax.ShapeDtypeStruct(g.shape, jnp.float32),
    )(g, smain, tails)


def kernel(_input, first_mat, second_mat):
    idx = _input.astype(jnp.int32).T
    g, tails = _gather_prod_sc(idx, first_mat, second_mat)
    smain = _sumsq_tc(first_mat, second_mat)
    return _combine_tc(g, smain, tails)


# speedup vs baseline: 1.3182x; 1.0247x over previous
import functools
import math

import jax
import jax.numpy as jnp
from jax import lax
from jax.experimental import pallas as pl
from jax.experimental.pallas import tpu as pltpu
from jax.experimental.pallas import tpu_sc as plsc

_K = 4
_SECOND_K = 4
_NIDX = _K + _SECOND_K
_BATCH = 16384
_TBL = 1000000
_LEVEL_SQ_SUM = math.exp(math.log(1e24) / _NIDX)
_SCALE_NUM = _LEVEL_SQ_SUM ** (_NIDX / 2.0)

_NC, _NS, _L = 2, 16, 16
_NW = _NC * _NS
_RPW = _BATCH // _NW

_CHUNK = 124928
_NCHUNK = 8
_MAIN = _CHUNK * _NCHUNK
_TAIL = _TBL - _MAIN


def _gather_prod_sc(idx, first_mat, second_mat):
    mesh = plsc.VectorSubcoreMesh(core_axis_name="c", subcore_axis_name="s")

    @functools.partial(
        pl.kernel,
        out_type=(jax.ShapeDtypeStruct((_BATCH,), jnp.float32),
                  jax.ShapeDtypeStruct((2 * _L,), jnp.float32)),
        mesh=mesh,
        scratch_types=[
            [pltpu.VMEM((_RPW,), jnp.int32) for _ in range(_NIDX)],
            [pltpu.VMEM((_RPW,), jnp.float32) for _ in range(_NIDX)],
            pltpu.VMEM((_RPW,), jnp.float32),
            pltpu.VMEM((_TAIL,), jnp.float32),
            pltpu.VMEM((_L,), jnp.float32),
            pltpu.SemaphoreType.DMA,
            pltpu.SemaphoreType.DMA,
        ],
    )
    def k(idx_hbm, fm_hbm, sm_hbm, out_hbm, tails_hbm,
          idxvs, valsvs, prodv, tailv, tpart, sem, isem):
        wid = lax.axis_index("s") * _NC + lax.axis_index("c")
        base = wid * _RPW
        icopies = [
            pltpu.async_copy(idx_hbm.at[j, pl.ds(base, _RPW)], idxvs[j], isem)
            for j in range(_NIDX)
        ]
        copies = []
        for j in range(_NIDX):
            icopies[j].wait()
            tbl = fm_hbm if j < _K else sm_hbm
            copies.append(pltpu.async_copy(tbl.at[idxvs[j]], valsvs[j], sem))

        for w, tbl in ((0, fm_hbm), (1, sm_hbm)):
            @pl.when(wid == w)
            def _():
                pltpu.sync_copy(tbl.at[pl.ds(_MAIN, _TAIL)], tailv)

                def tail_body(t, acc):
                    v = tailv[pl.ds(t * _L, _L)]
                    return acc + v * v

                tpart[...] = lax.fori_loop(
                    0, _TAIL // _L, tail_body, jnp.zeros((_L,), jnp.float32))
                pltpu.sync_copy(tpart, tails_hbm.at[pl.ds(w * _L, _L)])

        for cp in copies:
            cp.wait()

        def prod_body(i, carry):
            sl = pl.ds(i * _L, _L)
            p = valsvs[0][sl]
            for j in range(1, _NIDX):
                p = p * valsvs[j][sl]
            prodv[sl] = p
            return carry

        lax.fori_loop(0, _RPW // _L, prod_body, 0)
        pltpu.sync_copy(prodv, out_hbm.at[pl.ds(base, _RPW)])

    return k(idx, first_mat, second_mat)


def _sumsq_tc(fm, sm):
    def body(fm_hbm, sm_hbm, out_ref, fmv, smv, sem1, sem2):
        c1 = [pltpu.make_async_copy(fm_hbm.at[pl.ds(r * _CHUNK, _CHUNK)],
                                    fmv.at[r], sem1) for r in range(_NCHUNK)]
        c2 = [pltpu.make_async_copy(sm_hbm.at[pl.ds(r * _CHUNK, _CHUNK)],
                                    smv.at[r], sem2) for r in range(_NCHUNK)]
        for c in c1 + c2:
            c.start()
        for c in c1:
            c.wait()
        x = fmv[...]
        s1 = jnp.sum(x * x)
        for c in c2:
            c.wait()
        y = smv[...]
        s2 = jnp.sum(y * y)
        out_ref[...] = jnp.concatenate(
            [jnp.broadcast_to(s1, (1, 1)), jnp.broadcast_to(s2, (1, 1))], axis=1)

    return pl.pallas_call(
        body,
        in_specs=[pl.BlockSpec(memory_space=pltpu.MemorySpace.HBM),
                  pl.BlockSpec(memory_space=pltpu.MemorySpace.HBM)],
        out_specs=pl.BlockSpec(memory_space=pltpu.VMEM),
        out_shape=jax.ShapeDtypeStruct((1, 2), jnp.float32),
        scratch_shapes=[pltpu.VMEM((_NCHUNK, _CHUNK), jnp.float32),
                        pltpu.VMEM((_NCHUNK, _CHUNK), jnp.float32),
                        pltpu.SemaphoreType.DMA,
                        pltpu.SemaphoreType.DMA],
    )(fm, sm)


def _combine_tc(g, smain, tails):
    def body(g_ref, s_ref, t_ref, out_ref):
        s1 = s_ref[0, 0] + jnp.sum(t_ref[pl.ds(0, _L)])
        s2 = s_ref[0, 1] + jnp.sum(t_ref[pl.ds(_L, _L)])
        scale = _SCALE_NUM / ((s1 * s1) * (s2 * s2))
        out_ref[...] = g_ref[...] * scale

    return pl.pallas_call(
        body,
        out_shape=jax.ShapeDtypeStruct(g.shape, jnp.float32),
    )(g, smain, tails)


def kernel(_input, first_mat, second_mat):
    idx = _input.astype(jnp.int32).T
    g, tails = _gather_prod_sc(idx, first_mat, second_mat)
    smain = _sumsq_tc(first_mat, second_mat)
    return _combine_tc(g, smain, tails)
